# Initial kernel scaffold; baseline (speedup 1.0000x reference)
import jax
import jax.numpy as jnp
from jax.experimental import pallas as pl


def kernel(x, a_vals, W_F, edge_src, edge_dst):
    raise NotImplementedError("write your pallas kernel here")


# speedup vs baseline: 4.4312x; 4.4312x over previous
import functools

import jax
import jax.numpy as jnp
from jax import lax
from jax.experimental import pallas as pl
from jax.experimental.pallas import tpu as pltpu
from jax.experimental.pallas import tpu_sc as plsc

N = 10000
S = 2
D = 128
E = 320000

NC = 2
NS = 16
NW = NC * NS
C = 80
CH_PER_W = E // (NW * C)
STRIPE = 624
CPY = 208
TAIL = N - NS * STRIPE

BM = 1000


def _mm_body(x_ref, w_ref, o_ref):
    o_ref[...] = jnp.dot(x_ref[...], w_ref[0], preferred_element_type=jnp.float32)


def _matmul(x, w):
    return pl.pallas_call(
        _mm_body,
        grid=(S, N // BM),
        in_specs=[
            pl.BlockSpec((BM, D), lambda s, i: (i, 0)),
            pl.BlockSpec((1, D, D), lambda s, i: (s, 0, 0)),
        ],
        out_specs=pl.BlockSpec((BM, D), lambda s, i: (s * (N // BM) + i, 0)),
        out_shape=jax.ShapeDtypeStruct((S * N, D), jnp.float32),
    )(x, w)


_sc_mesh = plsc.VectorSubcoreMesh(core_axis_name="c", subcore_axis_name="s")


@functools.partial(
    pl.kernel,
    out_type=jax.ShapeDtypeStruct((NC, N, D), jnp.float32),
    mesh=_sc_mesh,
    scratch_types=[
        pltpu.VMEM((C,), jnp.int32),
        pltpu.VMEM((C,), jnp.int32),
        pltpu.VMEM((C,), jnp.float32),
        pltpu.VMEM((C, D), jnp.float32),
        pltpu.VMEM((CPY, D), jnp.float32),
        pltpu.VMEM_SHARED((N, D), jnp.float32),
        pltpu.SemaphoreType.DMA,
    ],
)
def _sc_edges(fw_hbm, src_hbm, dst_hbm, a_hbm, out_hbm,
              src_v, dst_v, a_v, rows_v, zb_v, acc_sh, sem):
    cid = lax.axis_index("c")
    sid = lax.axis_index("s")
    wid = cid * NS + sid

    zero16 = jnp.zeros((16,), jnp.float32)

    def _zrow(i, _):
        def _zcol(j, _):
            zb_v[i, pl.ds(j * 16, 16)] = zero16
            return 0
        return lax.fori_loop(0, D // 16, _zcol, 0)

    lax.fori_loop(0, CPY, _zrow, 0)
    row0 = sid * STRIPE
    for k in range(STRIPE // CPY):
        pltpu.sync_copy(zb_v, acc_sh.at[pl.ds(row0 + k * CPY, CPY)])

    @pl.when(sid == NS - 1)
    def _zero_tail():
        pltpu.sync_copy(zb_v.at[pl.ds(0, TAIL)],
                        acc_sh.at[pl.ds(NS * STRIPE, TAIL)])

    plsc.subcore_barrier()

    def _chunk(i, _):
        off = (wid * CH_PER_W + i) * C
        pltpu.sync_copy(src_hbm.at[pl.ds(off, C)], src_v)
        pltpu.sync_copy(dst_hbm.at[pl.ds(off, C)], dst_v)
        pltpu.sync_copy(a_hbm.at[pl.ds(off, C)], a_v)
        pltpu.async_copy(fw_hbm.at[src_v], rows_v, sem).wait()

        def _group(g, _):
            av = a_v[pl.ds(g * 16, 16)]
            base = g * 16
            for e in range(16):
                sp = av.at[jnp.full((16,), e, jnp.int32)].get(
                    mode="promise_in_bounds")
                r = base + e
                for j in range(D // 16):
                    rows_v[r, pl.ds(j * 16, 16)] = (
                        rows_v[r, pl.ds(j * 16, 16)] * sp)
            return 0

        lax.fori_loop(0, C // 16, _group, 0)
        pltpu.sync_copy(rows_v, acc_sh.at[dst_v], add=True)
        return 0

    lax.fori_loop(0, CH_PER_W, _chunk, 0)
    plsc.subcore_barrier()

    for k in range(STRIPE // CPY):
        r = row0 + k * CPY
        pltpu.sync_copy(acc_sh.at[pl.ds(r, CPY)], zb_v)
        pltpu.sync_copy(zb_v, out_hbm.at[cid, pl.ds(r, CPY)])

    @pl.when(sid == NS - 1)
    def _write_tail():
        pltpu.sync_copy(acc_sh.at[pl.ds(NS * STRIPE, TAIL)],
                        zb_v.at[pl.ds(0, TAIL)])
        pltpu.sync_copy(zb_v.at[pl.ds(0, TAIL)],
                        out_hbm.at[cid, pl.ds(NS * STRIPE, TAIL)])


def _add_body(p_ref, o_ref):
    o_ref[...] = p_ref[0] + p_ref[1]


def _combine(partials):
    return pl.pallas_call(
        _add_body,
        grid=(N // BM,),
        in_specs=[pl.BlockSpec((NC, BM, D), lambda i: (0, i, 0))],
        out_specs=pl.BlockSpec((BM, D), lambda i: (i, 0)),
        out_shape=jax.ShapeDtypeStruct((N, D), jnp.float32),
    )(partials)


def kernel(x, a_vals, W_F, edge_src, edge_dst):
    fw = _matmul(x, W_F)
    partials = _sc_edges(fw, edge_src, edge_dst, a_vals)
    return _combine(partials)


# speedup vs baseline: 10.1104x; 2.2816x over previous
import functools

import jax
import jax.numpy as jnp
from jax import lax
from jax.experimental import pallas as pl
from jax.experimental.pallas import tpu as pltpu
from jax.experimental.pallas import tpu_sc as plsc

N = 10000
S = 2
D = 128
E = 320000

NC = 2
NS = 16
NW = NC * NS
EPW = E // NW
C = 80
NCH = EPW // C
NB = 4
STRIPE = 624
ZROW = 80
TAIL = N - NS * STRIPE
CPY = 208

BM = 1000


def _mm_body(x_ref, w_ref, o_ref):
    o_ref[...] = jnp.dot(x_ref[...], w_ref[0], preferred_element_type=jnp.float32)


def _matmul(x, w):
    return pl.pallas_call(
        _mm_body,
        grid=(S, N // BM),
        in_specs=[
            pl.BlockSpec((BM, D), lambda s, i: (i, 0)),
            pl.BlockSpec((1, D, D), lambda s, i: (s, 0, 0)),
        ],
        out_specs=pl.BlockSpec((BM, D), lambda s, i: (s * (N // BM) + i, 0)),
        out_shape=jax.ShapeDtypeStruct((S * N, D), jnp.float32),
    )(x, w)


_sc_mesh = plsc.VectorSubcoreMesh(core_axis_name="c", subcore_axis_name="s")


@functools.partial(
    pl.kernel,
    out_type=jax.ShapeDtypeStruct((NC, N, D), jnp.float32),
    mesh=_sc_mesh,
    scratch_types=[
        [pltpu.VMEM((C,), jnp.int32)] * NB,
        [pltpu.VMEM((C,), jnp.int32)] * NB,
        [pltpu.VMEM((C,), jnp.float32)] * NB,
        [pltpu.VMEM((C, D), jnp.float32)] * NB,
        pltpu.VMEM_SHARED((N, D), jnp.float32),
        [pltpu.SemaphoreType.DMA] * NB,
        [pltpu.SemaphoreType.DMA] * NB,
        [pltpu.SemaphoreType.DMA] * NB,
    ],
)
def _sc_edges(fw_hbm, src_hbm, dst_hbm, a_hbm, out_hbm,
              srcs, dsts, avs, rows, acc_sh, gsem, ssem, isem):
    cid = lax.axis_index("c")
    sid = lax.axis_index("s")
    wid = cid * NS + sid
    ebase = wid * EPW

    def _issue_idx(i, sl):
        off = ebase + i * C
        pltpu.async_copy(src_hbm.at[pl.ds(off, C)], srcs[sl], isem[sl])
        pltpu.async_copy(dst_hbm.at[pl.ds(off, C)], dsts[sl], isem[sl])
        pltpu.async_copy(a_hbm.at[pl.ds(off, C)], avs[sl], isem[sl])

    def _wait_idx(sl):
        pltpu.make_async_copy(src_hbm.at[pl.ds(0, C)], srcs[sl], isem[sl]).wait()
        pltpu.make_async_copy(dst_hbm.at[pl.ds(0, C)], dsts[sl], isem[sl]).wait()
        pltpu.make_async_copy(a_hbm.at[pl.ds(0, C)], avs[sl], isem[sl]).wait()

    def _issue_gather(sl):
        pltpu.async_copy(fw_hbm.at[srcs[sl]], rows[sl], gsem[sl])

    def _wait_gather(sl):
        pltpu.make_async_copy(fw_hbm.at[srcs[sl]], rows[sl], gsem[sl]).wait()

    def _issue_scatter(sl):
        pltpu.async_copy(rows[sl], acc_sh.at[dsts[sl]], ssem[sl], add=True)

    def _wait_scatter(sl):
        pltpu.make_async_copy(rows[sl], acc_sh.at[dsts[sl]], ssem[sl]).wait()

    def _scale(sl):
        def _group(g, _):
            av = avs[sl][pl.ds(g * 16, 16)]
            for e in range(16):
                sp = av.at[jnp.full((16,), e, jnp.int32)].get(
                    mode="promise_in_bounds")
                r = g * 16 + e
                for j in range(D // 16):
                    rows[sl][r, pl.ds(j * 16, 16)] = (
                        rows[sl][r, pl.ds(j * 16, 16)] * sp)
            return 0

        lax.fori_loop(0, C // 16, _group, 0)

    def _step(i, u, wait_sc, next_gather, next_idx):
        _wait_gather(u)
        if wait_sc:
            _wait_scatter((u + 2) % NB)
        if next_gather:
            _wait_idx((u + 1) % NB)
            _issue_gather((u + 1) % NB)
        if next_idx:
            _issue_idx(i + 2, (u + 2) % NB)
        _scale(u)
        _issue_scatter(u)

    _issue_idx(0, 0)
    _issue_idx(1, 1)

    zero16 = jnp.zeros((16,), jnp.float32)

    def _zrow(i, _):
        def _zcol(j, _):
            rows[0][i, pl.ds(j * 16, 16)] = zero16
            return 0
        return lax.fori_loop(0, D // 16, _zcol, 0)

    lax.fori_loop(0, ZROW, _zrow, 0)
    row0 = sid * STRIPE
    for k in range(STRIPE // ZROW):
        pltpu.sync_copy(rows[0], acc_sh.at[pl.ds(row0 + k * ZROW, ZROW)])
    pltpu.sync_copy(rows[0].at[pl.ds(0, STRIPE % ZROW)],
                    acc_sh.at[pl.ds(row0 + (STRIPE // ZROW) * ZROW,
                                    STRIPE % ZROW)])

    @pl.when(sid == NS - 1)
    def _zero_tail():
        pltpu.sync_copy(rows[0].at[pl.ds(0, TAIL)],
                        acc_sh.at[pl.ds(NS * STRIPE, TAIL)])

    _wait_idx(0)
    _issue_gather(0)
    plsc.subcore_barrier()

    for i in range(NB):
        _step(i, i, wait_sc=(i >= 2), next_gather=True, next_idx=True)

    def _body(k, _):
        for u in range(NB):
            _step(4 * k + u, u, wait_sc=True, next_gather=True,
                  next_idx=True)
        return 0

    lax.fori_loop(1, (NCH - 5) // NB, _body, 0)

    i0 = NCH - 5
    for u in range(NB):
        _step(i0 + u, u, wait_sc=True, next_gather=True,
              next_idx=(i0 + u + 2 < NCH))
    _step(NCH - 1, 0, wait_sc=False, next_gather=False, next_idx=False)

    _wait_scatter(2)
    _wait_scatter(3)
    _wait_scatter(0)
    plsc.subcore_barrier()

    for k in range(STRIPE // CPY):
        r = row0 + k * CPY
        pltpu.sync_copy(acc_sh.at[pl.ds(r, CPY)], out_hbm.at[cid, pl.ds(r, CPY)])

    @pl.when(sid == NS - 1)
    def _write_tail():
        pltpu.sync_copy(acc_sh.at[pl.ds(NS * STRIPE, TAIL)],
                        out_hbm.at[cid, pl.ds(NS * STRIPE, TAIL)])


def _add_body(p_ref, o_ref):
    o_ref[...] = p_ref[0] + p_ref[1]


def _combine(partials):
    return pl.pallas_call(
        _add_body,
        grid=(N // BM,),
        in_specs=[pl.BlockSpec((NC, BM, D), lambda i: (0, i, 0))],
        out_specs=pl.BlockSpec((BM, D), lambda i: (i, 0)),
        out_shape=jax.ShapeDtypeStruct((N, D), jnp.float32),
    )(partials)


def kernel(x, a_vals, W_F, edge_src, edge_dst):
    fw = _matmul(x, W_F)
    partials = _sc_edges(fw, edge_src, edge_dst, a_vals)
    return _combine(partials)


# speedup vs baseline: 10.1126x; 1.0002x over previous
import functools

import jax
import jax.numpy as jnp
from jax import lax
from jax.experimental import pallas as pl
from jax.experimental.pallas import tpu as pltpu
from jax.experimental.pallas import tpu_sc as plsc

N = 10000
S = 2
D = 128
E = 320000

NC = 2
NS = 16
NW = NC * NS
EPW = E // NW
C = 80
NCH = EPW // C
NB = 4
STRIPE = 624
ZROW = 80
TAIL = N - NS * STRIPE
CPY = 208

BM = 1000


def _mm_body(x_ref, w_ref, o_ref):
    o_ref[...] = jnp.dot(x_ref[...], w_ref[0], preferred_element_type=jnp.float32)


def _matmul(x, w):
    return pl.pallas_call(
        _mm_body,
        grid=(S, N // BM),
        in_specs=[
            pl.BlockSpec((BM, D), lambda s, i: (i, 0)),
            pl.BlockSpec((1, D, D), lambda s, i: (s, 0, 0)),
        ],
        out_specs=pl.BlockSpec((BM, D), lambda s, i: (s * (N // BM) + i, 0)),
        out_shape=jax.ShapeDtypeStruct((S * N, D), jnp.float32),
    )(x, w)


_sc_mesh = plsc.VectorSubcoreMesh(core_axis_name="c", subcore_axis_name="s")


@functools.partial(
    pl.kernel,
    out_type=jax.ShapeDtypeStruct((NC, N, D), jnp.float32),
    mesh=_sc_mesh,
    scratch_types=[
        [pltpu.VMEM((C,), jnp.int32)] * NB,
        [pltpu.VMEM((C,), jnp.int32)] * NB,
        [pltpu.VMEM((C,), jnp.float32)] * NB,
        [pltpu.VMEM((C, D), jnp.float32)] * NB,
        pltpu.VMEM_SHARED((N, D), jnp.float32),
        [pltpu.SemaphoreType.DMA] * NB,
        [pltpu.SemaphoreType.DMA] * NB,
        [pltpu.SemaphoreType.DMA] * NB,
        pltpu.SemaphoreType.DMA,
    ],
)
def _sc_edges(fw_hbm, src_hbm, dst_hbm, a_hbm, out_hbm,
              srcs, dsts, avs, rows, acc_sh, gsem, ssem, isem, zsem):
    cid = lax.axis_index("c")
    sid = lax.axis_index("s")
    wid = cid * NS + sid
    ebase = wid * EPW

    def _issue_idx(i, sl):
        off = ebase + i * C
        pltpu.async_copy(src_hbm.at[pl.ds(off, C)], srcs[sl], isem[sl])
        pltpu.async_copy(dst_hbm.at[pl.ds(off, C)], dsts[sl], isem[sl])
        pltpu.async_copy(a_hbm.at[pl.ds(off, C)], avs[sl], isem[sl])

    def _wait_idx(sl):
        pltpu.make_async_copy(src_hbm.at[pl.ds(0, C)], srcs[sl], isem[sl]).wait()
        pltpu.make_async_copy(dst_hbm.at[pl.ds(0, C)], dsts[sl], isem[sl]).wait()
        pltpu.make_async_copy(a_hbm.at[pl.ds(0, C)], avs[sl], isem[sl]).wait()

    def _issue_gather(sl):
        pltpu.async_copy(fw_hbm.at[srcs[sl]], rows[sl], gsem[sl])

    def _wait_gather(sl):
        pltpu.make_async_copy(fw_hbm.at[srcs[sl]], rows[sl], gsem[sl]).wait()

    def _issue_scatter(sl):
        pltpu.async_copy(rows[sl], acc_sh.at[dsts[sl]], ssem[sl], add=True)

    def _wait_scatter(sl):
        pltpu.make_async_copy(rows[sl], acc_sh.at[dsts[sl]], ssem[sl]).wait()

    def _scale(sl):
        def _group(g, _):
            av = avs[sl][pl.ds(g * 16, 16)]
            for e in range(16):
                sp = av.at[jnp.full((16,), e, jnp.int32)].get(
                    mode="promise_in_bounds")
                r = g * 16 + e
                for j in range(D // 16):
                    rows[sl][r, pl.ds(j * 16, 16)] = (
                        rows[sl][r, pl.ds(j * 16, 16)] * sp)
            return 0

        lax.fori_loop(0, C // 16, _group, 0)

    def _step(i, u, wait_sc, next_gather, next_idx):
        _wait_gather(u)
        if wait_sc:
            _wait_scatter((u + 2) % NB)
        if next_gather:
            _wait_idx((u + 1) % NB)
            _issue_gather((u + 1) % NB)
        if next_idx:
            _issue_idx(i + 2, (u + 2) % NB)
        _scale(u)
        _issue_scatter(u)

    _issue_idx(0, 0)
    _issue_idx(1, 1)

    zero16 = jnp.zeros((16,), jnp.float32)

    def _zrow(i, _):
        def _zcol(j, _):
            rows[0][i, pl.ds(j * 16, 16)] = zero16
            return 0
        return lax.fori_loop(0, D // 16, _zcol, 0)

    lax.fori_loop(0, ZROW, _zrow, 0)
    row0 = sid * STRIPE
    for k in range(STRIPE // ZROW):
        pltpu.async_copy(rows[0], acc_sh.at[pl.ds(row0 + k * ZROW, ZROW)],
                         zsem)
    pltpu.async_copy(rows[0].at[pl.ds(0, STRIPE % ZROW)],
                     acc_sh.at[pl.ds(row0 + (STRIPE // ZROW) * ZROW,
                                     STRIPE % ZROW)], zsem)

    @pl.when(sid == NS - 1)
    def _zero_tail():
        pltpu.async_copy(rows[0].at[pl.ds(0, TAIL)],
                         acc_sh.at[pl.ds(NS * STRIPE, TAIL)], zsem)

    _wait_idx(0)
    for k in range(STRIPE // ZROW):
        pltpu.make_async_copy(
            rows[0], acc_sh.at[pl.ds(row0 + k * ZROW, ZROW)], zsem).wait()
    pltpu.make_async_copy(
        rows[0].at[pl.ds(0, STRIPE % ZROW)],
        acc_sh.at[pl.ds(row0, STRIPE % ZROW)], zsem).wait()

    @pl.when(sid == NS - 1)
    def _wait_zero_tail():
        pltpu.make_async_copy(rows[0].at[pl.ds(0, TAIL)],
                              acc_sh.at[pl.ds(NS * STRIPE, TAIL)],
                              zsem).wait()

    _issue_gather(0)
    plsc.subcore_barrier()

    for i in range(NB):
        _step(i, i, wait_sc=(i >= 2), next_gather=True, next_idx=True)

    def _body(k, _):
        for u in range(NB):
            _step(4 * k + u, u, wait_sc=True, next_gather=True,
                  next_idx=True)
        return 0

    lax.fori_loop(1, (NCH - 5) // NB, _body, 0)

    i0 = NCH - 5
    for u in range(NB):
        _step(i0 + u, u, wait_sc=True, next_gather=True,
              next_idx=(i0 + u + 2 < NCH))
    _step(NCH - 1, 0, wait_sc=False, next_gather=False, next_idx=False)

    _wait_scatter(2)
    _wait_scatter(3)
    _wait_scatter(0)
    plsc.subcore_barrier()

    for k in range(STRIPE // CPY):
        r = row0 + k * CPY
        pltpu.async_copy(acc_sh.at[pl.ds(r, CPY)],
                         out_hbm.at[cid, pl.ds(r, CPY)], zsem)

    @pl.when(sid == NS - 1)
    def _write_tail():
        pltpu.async_copy(acc_sh.at[pl.ds(NS * STRIPE, TAIL)],
                         out_hbm.at[cid, pl.ds(NS * STRIPE, TAIL)], zsem)

    for k in range(STRIPE // CPY):
        r = row0 + k * CPY
        pltpu.make_async_copy(acc_sh.at[pl.ds(r, CPY)],
                              out_hbm.at[cid, pl.ds(r, CPY)], zsem).wait()

    @pl.when(sid == NS - 1)
    def _wait_write_tail():
        pltpu.make_async_copy(acc_sh.at[pl.ds(NS * STRIPE, TAIL)],
                              out_hbm.at[cid, pl.ds(NS * STRIPE, TAIL)],
                              zsem).wait()


def _add_body(p_ref, o_ref):
    o_ref[...] = p_ref[0] + p_ref[1]


def _combine(partials):
    return pl.pallas_call(
        _add_body,
        grid=(N // BM,),
        in_specs=[pl.BlockSpec((NC, BM, D), lambda i: (0, i, 0))],
        out_specs=pl.BlockSpec((BM, D), lambda i: (i, 0)),
        out_shape=jax.ShapeDtypeStruct((N, D), jnp.float32),
    )(partials)


def kernel(x, a_vals, W_F, edge_src, edge_dst):
    fw = _matmul(x, W_F)
    partials = _sc_edges(fw, edge_src, edge_dst, a_vals)
    return _combine(partials)


# speedup vs baseline: 12.5654x; 1.2425x over previous
import functools

import jax
import jax.numpy as jnp
from jax import lax
from jax.experimental import pallas as pl
from jax.experimental.pallas import tpu as pltpu
from jax.experimental.pallas import tpu_sc as plsc

N = 10000
S = 2
D = 128
E = 320000

NC = 2
NS = 16
NW = NC * NS
EPW = E // NW
C = 80
NCH = EPW // C
NB = 4
NI = 8
STRIPE = 624
ZROW = 80
TAIL = N - NS * STRIPE
CPY = 208

BM = 1000


def _mm_body(x_ref, w_ref, o_ref):
    o_ref[...] = jnp.dot(x_ref[...], w_ref[0], preferred_element_type=jnp.float32)


def _matmul(x, w):
    return pl.pallas_call(
        _mm_body,
        grid=(S, N // BM),
        in_specs=[
            pl.BlockSpec((BM, D), lambda s, i: (i, 0)),
            pl.BlockSpec((1, D, D), lambda s, i: (s, 0, 0)),
        ],
        out_specs=pl.BlockSpec((BM, D), lambda s, i: (s * (N // BM) + i, 0)),
        out_shape=jax.ShapeDtypeStruct((S * N, D), jnp.float32),
    )(x, w)


_sc_mesh = plsc.VectorSubcoreMesh(core_axis_name="c", subcore_axis_name="s")


@functools.partial(
    pl.kernel,
    out_type=jax.ShapeDtypeStruct((NC, N, D), jnp.float32),
    mesh=_sc_mesh,
    scratch_types=[
        [pltpu.VMEM((C,), jnp.int32)] * NI,
        [pltpu.VMEM((C,), jnp.int32)] * NI,
        [pltpu.VMEM((C,), jnp.float32)] * NI,
        [pltpu.VMEM((C, D), jnp.float32)] * NB,
        pltpu.VMEM_SHARED((N, D), jnp.float32),
        [pltpu.SemaphoreType.DMA] * NB,
        [pltpu.SemaphoreType.DMA] * NB,
        [pltpu.SemaphoreType.DMA] * NI,
        pltpu.SemaphoreType.DMA,
    ],
)
def _sc_edges(fw_hbm, src_hbm, dst_hbm, a_hbm, out_hbm,
              srcs, dsts, avs, rows, acc_sh, gsem, ssem, isem, zsem):
    cid = lax.axis_index("c")
    sid = lax.axis_index("s")
    wid = cid * NS + sid
    ebase = wid * EPW

    def _issue_idx(i, sl):
        off = ebase + i * C
        pltpu.async_copy(src_hbm.at[pl.ds(off, C)], srcs[sl], isem[sl])
        pltpu.async_copy(dst_hbm.at[pl.ds(off, C)], dsts[sl], isem[sl])
        pltpu.async_copy(a_hbm.at[pl.ds(off, C)], avs[sl], isem[sl])

    def _wait_idx(sl):
        pltpu.make_async_copy(src_hbm.at[pl.ds(0, C)], srcs[sl], isem[sl]).wait()
        pltpu.make_async_copy(dst_hbm.at[pl.ds(0, C)], dsts[sl], isem[sl]).wait()
        pltpu.make_async_copy(a_hbm.at[pl.ds(0, C)], avs[sl], isem[sl]).wait()

    def _issue_gather(isl, rsl):
        pltpu.async_copy(fw_hbm.at[srcs[isl]], rows[rsl], gsem[rsl])

    def _wait_gather(isl, rsl):
        pltpu.make_async_copy(fw_hbm.at[srcs[isl]], rows[rsl],
                              gsem[rsl]).wait()

    def _issue_scatter(isl, rsl):
        pltpu.async_copy(rows[rsl], acc_sh.at[dsts[isl]], ssem[rsl], add=True)

    def _wait_scatter(isl, rsl):
        pltpu.make_async_copy(rows[rsl], acc_sh.at[dsts[isl]],
                              ssem[rsl]).wait()

    def _scale(isl, rsl):
        def _group(g, _):
            av = avs[isl][pl.ds(g * 16, 16)]
            for e in range(16):
                sp = av.at[jnp.full((16,), e, jnp.int32)].get(
                    mode="promise_in_bounds")
                r = g * 16 + e
                for j in range(D // 16):
                    rows[rsl][r, pl.ds(j * 16, 16)] = (
                        rows[rsl][r, pl.ds(j * 16, 16)] * sp)
            return 0

        lax.fori_loop(0, C // 16, _group, 0)

    for i in range(4):
        _issue_idx(i, i)

    zero16 = jnp.zeros((16,), jnp.float32)

    def _zrow(i, _):
        def _zcol(j, _):
            rows[0][i, pl.ds(j * 16, 16)] = zero16
            return 0
        return lax.fori_loop(0, D // 16, _zcol, 0)

    lax.fori_loop(0, ZROW, _zrow, 0)
    row0 = sid * STRIPE
    for k in range(STRIPE // ZROW):
        pltpu.async_copy(rows[0], acc_sh.at[pl.ds(row0 + k * ZROW, ZROW)],
                         zsem)
    pltpu.async_copy(rows[0].at[pl.ds(0, STRIPE % ZROW)],
                     acc_sh.at[pl.ds(row0 + (STRIPE // ZROW) * ZROW,
                                     STRIPE % ZROW)], zsem)

    @pl.when(sid == NS - 1)
    def _zero_tail():
        pltpu.async_copy(rows[0].at[pl.ds(0, TAIL)],
                         acc_sh.at[pl.ds(NS * STRIPE, TAIL)], zsem)

    for k in range(STRIPE // ZROW):
        pltpu.make_async_copy(
            rows[0], acc_sh.at[pl.ds(row0 + k * ZROW, ZROW)], zsem).wait()
    pltpu.make_async_copy(
        rows[0].at[pl.ds(0, STRIPE % ZROW)],
        acc_sh.at[pl.ds(row0, STRIPE % ZROW)], zsem).wait()

    @pl.when(sid == NS - 1)
    def _wait_zero_tail():
        pltpu.make_async_copy(rows[0].at[pl.ds(0, TAIL)],
                              acc_sh.at[pl.ds(NS * STRIPE, TAIL)],
                              zsem).wait()

    _wait_idx(0)
    _issue_gather(0, 0)
    _wait_idx(1)
    _issue_gather(1, 1)
    plsc.subcore_barrier()

    def _body(k, _):
        i0 = 8 * k
        for u in range(NI):
            i = i0 + u
            u4 = u % 4

            @pl.when(i < NCH)
            def _proc():
                _wait_gather(u, u4)

                @pl.when(i >= 2)
                def _free():
                    _wait_scatter((u + 6) % NI, (u4 + 2) % NB)

                @pl.when(i + 2 < NCH)
                def _next_gather():
                    _wait_idx((u + 2) % NI)
                    _issue_gather((u + 2) % NI, (u4 + 2) % NB)

                @pl.when(i + 4 < NCH)
                def _next_idx():
                    _issue_idx(i + 4, (u + 4) % NI)

                _scale(u, u4)
                _issue_scatter(u, u4)

        return 0

    lax.fori_loop(0, (NCH + 7) // 8, _body, 0)

    _wait_scatter((NCH - 2) % NI, (NCH - 2) % NB)
    _wait_scatter((NCH - 1) % NI, (NCH - 1) % NB)
    plsc.subcore_barrier()

    for k in range(STRIPE // CPY):
        r = row0 + k * CPY
        pltpu.async_copy(acc_sh.at[pl.ds(r, CPY)],
                         out_hbm.at[cid, pl.ds(r, CPY)], zsem)

    @pl.when(sid == NS - 1)
    def _write_tail():
        pltpu.async_copy(acc_sh.at[pl.ds(NS * STRIPE, TAIL)],
                         out_hbm.at[cid, pl.ds(NS * STRIPE, TAIL)], zsem)

    for k in range(STRIPE // CPY):
        r = row0 + k * CPY
        pltpu.make_async_copy(acc_sh.at[pl.ds(r, CPY)],
                              out_hbm.at[cid, pl.ds(r, CPY)], zsem).wait()

    @pl.when(sid == NS - 1)
    def _wait_write_tail():
        pltpu.make_async_copy(acc_sh.at[pl.ds(NS * STRIPE, TAIL)],
                              out_hbm.at[cid, pl.ds(NS * STRIPE, TAIL)],
                              zsem).wait()


def _add_body(p_ref, o_ref):
    o_ref[...] = p_ref[0] + p_ref[1]


def _combine(partials):
    return pl.pallas_call(
        _add_body,
        grid=(N // BM,),
        in_specs=[pl.BlockSpec((NC, BM, D), lambda i: (0, i, 0))],
        out_specs=pl.BlockSpec((BM, D), lambda i: (i, 0)),
        out_shape=jax.ShapeDtypeStruct((N, D), jnp.float32),
    )(partials)


def kernel(x, a_vals, W_F, edge_src, edge_dst):
    fw = _matmul(x, W_F)
    partials = _sc_edges(fw, edge_src, edge_dst, a_vals)
    return _combine(partials)
